# baseline (speedup 1.0000x reference)
import jax
import jax.numpy as jnp
from jax.experimental import pallas as pl

INPUT_SIZE = 65536
SIZE = 128
STRIDE = 16
NPZ = 64
K = 8
NUM_ZONES = (INPUT_SIZE - (SIZE - 1)) // STRIDE

ZB = 56
NB = NUM_ZONES // ZB


def _body(win_ref, w_ref, out_ref):
    w = w_ref[...].astype(jnp.bfloat16).astype(jnp.float32)
    win = win_ref[...].astype(jnp.bfloat16).astype(jnp.float32)
    resp = jnp.sum(w * win[:, None, :], axis=-1)
    work = resp
    for _ in range(K - 1):
        m = jnp.max(work, axis=-1, keepdims=True)
        work = jnp.where(work == m, -jnp.inf, work)
    thresh = jnp.max(work, axis=-1, keepdims=True)
    out_ref[...] = jnp.where(resp >= thresh, resp, jnp.zeros_like(resp))


def kernel(x, W):
    xf = jnp.reshape(x, (-1,))
    idx = jnp.arange(NUM_ZONES)[:, None] * STRIDE + jnp.arange(SIZE)[None, :]
    windows = jnp.take(xf, idx, axis=0)
    return pl.pallas_call(
        _body,
        grid=(NB,),
        in_specs=[
            pl.BlockSpec((ZB, SIZE), lambda i: (i, 0)),
            pl.BlockSpec((ZB, NPZ, SIZE), lambda i: (i, 0, 0)),
        ],
        out_specs=pl.BlockSpec((ZB, NPZ), lambda i: (i, 0)),
        out_shape=jax.ShapeDtypeStruct((NUM_ZONES, NPZ), jnp.float32),
    )(windows, W)


# speedup vs baseline: 1.0787x; 1.0787x over previous
import jax
import jax.numpy as jnp
from jax import lax
from jax.experimental import pallas as pl

INPUT_SIZE = 65536
SIZE = 128
STRIDE = 16
NPZ = 64
K = 8
NUM_ZONES = (INPUT_SIZE - (SIZE - 1)) // STRIDE

ZB = 56
NB = NUM_ZONES // ZB


def _body(win_ref, w_ref, out_ref):
    winT = win_ref[...].astype(jnp.bfloat16).T
    cols = []
    for z in range(ZB):
        wz = w_ref[z].astype(jnp.bfloat16)
        cols.append(
            lax.dot_general(wz, winT[:, z:z + 1],
                            (((1,), (0,)), ((), ())),
                            preferred_element_type=jnp.float32))
    respT = jnp.concatenate(cols, axis=1)
    work = respT
    for _ in range(K - 1):
        m = jnp.max(work, axis=0, keepdims=True)
        work = jnp.where(work == m, -jnp.inf, work)
    thresh = jnp.max(work, axis=0, keepdims=True)
    outT = jnp.where(respT >= thresh, respT, jnp.zeros_like(respT))
    out_ref[...] = outT.T


def kernel(x, W):
    xf = jnp.reshape(x, (-1,))
    idx = jnp.arange(NUM_ZONES)[:, None] * STRIDE + jnp.arange(SIZE)[None, :]
    windows = jnp.take(xf, idx, axis=0)
    return pl.pallas_call(
        _body,
        grid=(NB,),
        in_specs=[
            pl.BlockSpec((ZB, SIZE), lambda i: (i, 0)),
            pl.BlockSpec((ZB, NPZ, SIZE), lambda i: (i, 0, 0)),
        ],
        out_specs=pl.BlockSpec((ZB, NPZ), lambda i: (i, 0)),
        out_shape=jax.ShapeDtypeStruct((NUM_ZONES, NPZ), jnp.float32),
    )(windows, W)


# speedup vs baseline: 40.8264x; 37.8478x over previous
import jax
import jax.numpy as jnp
from jax import lax
from jax.experimental import pallas as pl

INPUT_SIZE = 65536
SIZE = 128
STRIDE = 16
NPZ = 64
K = 8
NUM_ZONES = (INPUT_SIZE - (SIZE - 1)) // STRIDE

ZB = 64
NB = (NUM_ZONES + ZB - 1) // ZB
XROWS = INPUT_SIZE // SIZE


def _body(xm_ref, xh_ref, w_ref, out_ref):
    xa = jnp.concatenate([xm_ref[...], xh_ref[...]], axis=0)
    b = jnp.roll(xa, -1, axis=0)
    lane = lax.broadcasted_iota(jnp.int32, (16, SIZE), 1)
    rows = []
    for r in range(8):
        if r == 0:
            rr = xa
        else:
            rl = jnp.roll(xa, -16 * r, axis=1)
            rlb = jnp.roll(b, -16 * r, axis=1)
            rr = jnp.where(lane < SIZE - 16 * r, rl, rlb)
        rows.append(rr[:8])
    win = jnp.stack(rows, axis=1).reshape(ZB, SIZE)
    winT = win.T.astype(jnp.bfloat16)
    cols = []
    for z in range(ZB):
        wz = w_ref[z].astype(jnp.bfloat16)
        cols.append(
            lax.dot_general(wz, winT[:, z:z + 1],
                            (((1,), (0,)), ((), ())),
                            preferred_element_type=jnp.float32))
    respT = jnp.concatenate(cols, axis=1)
    work = respT
    for _ in range(K - 1):
        m = jnp.max(work, axis=0, keepdims=True)
        work = jnp.where(work == m, -jnp.inf, work)
    thresh = jnp.max(work, axis=0, keepdims=True)
    outT = jnp.where(respT >= thresh, respT, jnp.zeros_like(respT))
    out_ref[...] = outT.T


def kernel(x, W):
    xs = jnp.reshape(x, (XROWS, SIZE))
    return pl.pallas_call(
        _body,
        grid=(NB,),
        in_specs=[
            pl.BlockSpec((8, SIZE), lambda i: (i, 0)),
            pl.BlockSpec((8, SIZE), lambda i: (jnp.minimum(i + 1, NB - 1), 0)),
            pl.BlockSpec((ZB, NPZ, SIZE), lambda i: (i, 0, 0)),
        ],
        out_specs=pl.BlockSpec((ZB, NPZ), lambda i: (i, 0)),
        out_shape=jax.ShapeDtypeStruct((NUM_ZONES, NPZ), jnp.float32),
    )(xs, xs, W)


# speedup vs baseline: 63.3284x; 1.5512x over previous
import jax
import jax.numpy as jnp
from jax import lax
from jax.experimental import pallas as pl

INPUT_SIZE = 65536
SIZE = 128
STRIDE = 16
NPZ = 64
K = 8
NUM_ZONES = (INPUT_SIZE - (SIZE - 1)) // STRIDE

ZB = 128
NB = (NUM_ZONES + ZB - 1) // ZB
XROWS = INPUT_SIZE // SIZE


def _body(xm_ref, xh_ref, w_ref, out_ref):
    xa = jnp.concatenate([xm_ref[...], xh_ref[...]], axis=0)
    b = jnp.roll(xa, -1, axis=0)
    lane = lax.broadcasted_iota(jnp.int32, (ZB // 8 + 8, SIZE), 1)
    rows = []
    for r in range(8):
        if r == 0:
            rr = xa
        else:
            rl = jnp.roll(xa, -16 * r, axis=1)
            rlb = jnp.roll(b, -16 * r, axis=1)
            rr = jnp.where(lane < SIZE - 16 * r, rl, rlb)
        rows.append(rr[:ZB // 8])
    win = jnp.stack(rows, axis=1).reshape(ZB, SIZE)
    win = win.astype(jnp.bfloat16)
    resp = lax.dot_general(w_ref[...].astype(jnp.bfloat16), win,
                           (((2,), (1,)), ((0,), (0,))),
                           preferred_element_type=jnp.float32)
    respT = resp.T
    work = respT
    for _ in range(K - 1):
        m = jnp.max(work, axis=0, keepdims=True)
        work = jnp.where(work == m, -jnp.inf, work)
    thresh = jnp.max(work, axis=0, keepdims=True)
    outT = jnp.where(respT >= thresh, respT, jnp.zeros_like(respT))
    out_ref[...] = outT.T


def kernel(x, W):
    xs = jnp.reshape(x, (XROWS, SIZE))
    return pl.pallas_call(
        _body,
        grid=(NB,),
        in_specs=[
            pl.BlockSpec((ZB // 8, SIZE), lambda i: (i, 0)),
            pl.BlockSpec((8, SIZE),
                         lambda i: (jnp.minimum((ZB // 64) * (i + 1),
                                                XROWS // 8 - 1), 0)),
            pl.BlockSpec((ZB, NPZ, SIZE), lambda i: (i, 0, 0)),
        ],
        out_specs=pl.BlockSpec((ZB, NPZ), lambda i: (i, 0)),
        out_shape=jax.ShapeDtypeStruct((NUM_ZONES, NPZ), jnp.float32),
    )(xs, xs, W)
